# Initial kernel scaffold; baseline (speedup 1.0000x reference)
import jax
import jax.numpy as jnp
from jax.experimental import pallas as pl


def kernel(queries, control_points, tx, ty, tz):
    raise NotImplementedError("write your pallas kernel here")


# speedup vs baseline: 68.3852x; 68.3852x over previous
import functools

import jax
import jax.numpy as jnp
from jax import lax
from jax.experimental import pallas as pl
from jax.experimental.pallas import tpu as pltpu
from jax.experimental.pallas import tpu_sc as plsc

NQ = 131072
GRID = 128
NSEG = GRID - 3
NKNOT = GRID + 4
NGROUP = GRID * GRID * GRID // 4
NCORE = 2
NSUB = 16
NW = NCORE * NSUB
QPW = NQ // NW
BQ = 64
NV = BQ // 16
NSTEP = QPW // BQ


def _splat(v, dtype=jnp.int32):
    return jnp.full((16,), v, dtype)


def _basis(tv, x):
    xi = x * jnp.float32(NSEG)
    j0 = jnp.clip(xi.astype(jnp.int32), 0, NSEG - 1)
    ta = plsc.load_gather(tv, [j0 + 3])
    tb = plsc.load_gather(tv, [j0 + 4])
    j = j0 + (x >= tb).astype(jnp.int32) - (x < ta).astype(jnp.int32)
    j = jnp.clip(j, 0, NSEG - 1)
    t1 = plsc.load_gather(tv, [j + 1])
    t2 = plsc.load_gather(tv, [j + 2])
    t3 = plsc.load_gather(tv, [j + 3])
    t4 = plsc.load_gather(tv, [j + 4])
    t5 = plsc.load_gather(tv, [j + 5])
    t6 = plsc.load_gather(tv, [j + 6])
    l1 = x - t3
    l2 = x - t2
    l3 = x - t1
    r1 = t4 - x
    r2 = t5 - x
    r3 = t6 - x
    tmp = jnp.float32(1.0) / (r1 + l1)
    n0 = r1 * tmp
    n1 = l1 * tmp
    tmp = n0 / (r1 + l2)
    n0 = r1 * tmp
    sv = l2 * tmp
    tmp = n1 / (r2 + l1)
    n1 = sv + r2 * tmp
    n2 = l1 * tmp
    tmp = n0 / (r1 + l3)
    n0 = r1 * tmp
    sv = l3 * tmp
    tmp = n1 / (r2 + l2)
    n1 = sv + r2 * tmp
    sv = l2 * tmp
    tmp = n2 / (r3 + l1)
    n2 = sv + r3 * tmp
    n3 = l1 * tmp
    return j, n0, n1, n2, n3


def _body(q_hbm, cp_hbm, tx_hbm, ty_hbm, tz_hbm, out_hbm,
          txv, tyv, tzv, q_v, idx_v, g_v, w_v, s_v, o_v, sem):
    cid = lax.axis_index("c")
    sid = lax.axis_index("s")
    wid = sid * NCORE + cid
    qbase = wid * QPW
    pltpu.sync_copy(tx_hbm, txv)
    pltpu.sync_copy(ty_hbm, tyv)
    pltpu.sync_copy(tz_hbm, tzv)
    lanes = lax.iota(jnp.int32, 16)

    @pl.loop(0, NSTEP)
    def _step(si):
        qb = qbase + si * BQ
        pltpu.sync_copy(q_hbm.at[pl.ds(qb, BQ)], q_v)
        for v in range(NV):
            iq = lanes + (v * 16)
            qx = plsc.load_gather(q_v, [iq, _splat(0)])
            qy = plsc.load_gather(q_v, [iq, _splat(1)])
            qz = plsc.load_gather(q_v, [iq, _splat(2)])
            jx, bx0, bx1, bx2, bx3 = _basis(txv, qx)
            jy, by0, by1, by2, by3 = _basis(tyv, qy)
            jz, bz0, bz1, bz2, bz3 = _basis(tzv, qz)
            allb = (bx0, bx1, bx2, bx3, by0, by1, by2, by3, bz0, bz1, bz2, bz3)
            for r, n in enumerate(allb):
                w_v[r, pl.ds(v * 16, 16)] = n
            s_v[pl.ds(v * 16, 16)] = (jx & 3) * 3
            gbase = (jx >> 2) + jy * 32 + jz * 4096
            q2 = iq * 2
            for m in range(16):
                my = m & 3
                mz = m >> 2
                g0 = gbase + (32 * my + 4096 * mz)
                g1 = jnp.minimum(g0 + 1, NGROUP - 1)
                plsc.store_scatter(idx_v, [_splat(m), q2], g0)
                plsc.store_scatter(idx_v, [_splat(m), q2 + 1], g1)
        handles = [
            pltpu.async_copy(cp_hbm.at[idx_v.at[m]], g_v.at[m], sem)
            for m in range(16)
        ]
        for h in handles:
            h.wait()
        for v in range(NV):
            iq = lanes + (v * 16)
            s3 = s_v[pl.ds(v * 16, 16)]
            q2 = iq * 2
            bx = [w_v[r, pl.ds(v * 16, 16)] for r in range(4)]
            by = [w_v[4 + r, pl.ds(v * 16, 16)] for r in range(4)]
            bz = [w_v[8 + r, pl.ds(v * 16, 16)] for r in range(4)]
            d1 = {}
            d2 = {}
            for mx in range(4):
                for c in range(3):
                    u = s3 + (mx * 3 + c)
                    ge = (u >= 12).astype(jnp.int32)
                    d1[(mx, c)] = q2 + ge
                    d2[(mx, c)] = u - ge * 12
            acc = [_splat(0.0, jnp.float32) for _ in range(3)]
            for m in range(16):
                wyz = by[m & 3] * bz[m >> 2]
                for mx in range(4):
                    w = wyz * bx[mx]
                    for c in range(3):
                        gval = plsc.load_gather(
                            g_v, [_splat(m), d1[(mx, c)], d2[(mx, c)]])
                        acc[c] = acc[c] + w * gval
            for c in range(3):
                plsc.store_scatter(o_v, [iq, _splat(c)], acc[c])
        pltpu.sync_copy(o_v, out_hbm.at[pl.ds(qb, BQ)])


@jax.jit
def kernel(queries, control_points, tx, ty, tz):
    cpv = jnp.pad(control_points.reshape(NGROUP, 12), ((0, 0), (0, 4)))
    mesh = plsc.VectorSubcoreMesh(
        core_axis_name="c", subcore_axis_name="s",
        num_cores=NCORE, num_subcores=NSUB)
    spline = pl.kernel(
        _body,
        out_type=jax.ShapeDtypeStruct((NQ, 3), jnp.float32),
        mesh=mesh,
        compiler_params=pltpu.CompilerParams(
            needs_layout_passes=False, use_tc_tiling_on_sc=False),
        scratch_types=[
            pltpu.VMEM((NKNOT,), jnp.float32),
            pltpu.VMEM((NKNOT,), jnp.float32),
            pltpu.VMEM((NKNOT,), jnp.float32),
            pltpu.VMEM((BQ, 3), jnp.float32),
            pltpu.VMEM((16, 2 * BQ), jnp.int32),
            pltpu.VMEM((16, 2 * BQ, 16), jnp.float32),
            pltpu.VMEM((12, BQ), jnp.float32),
            pltpu.VMEM((BQ,), jnp.int32),
            pltpu.VMEM((BQ, 3), jnp.float32),
            pltpu.SemaphoreType.DMA,
        ],
    )
    return spline(queries, cpv, tx, ty, tz)


# speedup vs baseline: 73.7090x; 1.0778x over previous
import functools

import jax
import jax.numpy as jnp
from jax import lax
from jax.experimental import pallas as pl
from jax.experimental.pallas import tpu as pltpu
from jax.experimental.pallas import tpu_sc as plsc

NQ = 131072
GRID = 128
NSEG = GRID - 3
NKNOT = GRID + 4
NROW = GRID * GRID * GRID * 3 // 16
NCORE = 2
NSUB = 16
NW = NCORE * NSUB
QPW = NQ // NW
BQ = 64
NV = BQ // 16
NSTEP = QPW // BQ


def _splat(v, dtype=jnp.int32):
    return jnp.full((16,), v, dtype)


def _basis(tv, x):
    xi = x * jnp.float32(NSEG)
    j0 = jnp.clip(xi.astype(jnp.int32), 0, NSEG - 1)
    ta = plsc.load_gather(tv, [j0 + 3])
    tb = plsc.load_gather(tv, [j0 + 4])
    j = j0 + (x >= tb).astype(jnp.int32) - (x < ta).astype(jnp.int32)
    j = jnp.clip(j, 0, NSEG - 1)
    t1 = plsc.load_gather(tv, [j + 1])
    t2 = plsc.load_gather(tv, [j + 2])
    t3 = plsc.load_gather(tv, [j + 3])
    t4 = plsc.load_gather(tv, [j + 4])
    t5 = plsc.load_gather(tv, [j + 5])
    t6 = plsc.load_gather(tv, [j + 6])
    l1 = x - t3
    l2 = x - t2
    l3 = x - t1
    r1 = t4 - x
    r2 = t5 - x
    r3 = t6 - x
    tmp = jnp.float32(1.0) / (r1 + l1)
    n0 = r1 * tmp
    n1 = l1 * tmp
    tmp = n0 / (r1 + l2)
    n0 = r1 * tmp
    sv = l2 * tmp
    tmp = n1 / (r2 + l1)
    n1 = sv + r2 * tmp
    n2 = l1 * tmp
    tmp = n0 / (r1 + l3)
    n0 = r1 * tmp
    sv = l3 * tmp
    tmp = n1 / (r2 + l2)
    n1 = sv + r2 * tmp
    sv = l2 * tmp
    tmp = n2 / (r3 + l1)
    n2 = sv + r3 * tmp
    n3 = l1 * tmp
    return j, n0, n1, n2, n3


def _body(q_hbm, cp_hbm, tx_hbm, ty_hbm, tz_hbm, out_hbm,
          txv, tyv, tzv, q_v, idx_v, g_v, w_v, s_v, o_v, sem):
    cid = lax.axis_index("c")
    sid = lax.axis_index("s")
    wid = sid * NCORE + cid
    qbase = wid * QPW
    pltpu.sync_copy(tx_hbm, txv)
    pltpu.sync_copy(ty_hbm, tyv)
    pltpu.sync_copy(tz_hbm, tzv)
    lanes = lax.iota(jnp.int32, 16)

    @pl.loop(0, NSTEP)
    def _step(si):
        qb = qbase + si * BQ
        pltpu.sync_copy(q_hbm.at[pl.ds(qb, BQ)], q_v)
        for v in range(NV):
            iq = lanes + (v * 16)
            qx = plsc.load_gather(q_v, [iq, _splat(0)])
            qy = plsc.load_gather(q_v, [iq, _splat(1)])
            qz = plsc.load_gather(q_v, [iq, _splat(2)])
            jx, bx0, bx1, bx2, bx3 = _basis(txv, qx)
            jy, by0, by1, by2, by3 = _basis(tyv, qy)
            jz, bz0, bz1, bz2, bz3 = _basis(tzv, qz)
            allb = (bx0, bx1, bx2, bx3, by0, by1, by2, by3, bz0, bz1, bz2, bz3)
            for r, n in enumerate(allb):
                w_v[r, pl.ds(v * 16, 16)] = n
            base3 = jx * 3 + jy * 384 + jz * 49152
            s_v[pl.ds(v * 16, 16)] = base3 & 15
            gbase = base3 >> 4
            q2 = iq * 2
            for m in range(16):
                my = m & 3
                mz = m >> 2
                g0 = gbase + (24 * my + 3072 * mz)
                g1 = jnp.minimum(g0 + 1, NROW - 1)
                plsc.store_scatter(idx_v, [_splat(m), q2], g0)
                plsc.store_scatter(idx_v, [_splat(m), q2 + 1], g1)
        handles = [
            pltpu.async_copy(cp_hbm.at[idx_v.at[m]], g_v.at[m], sem)
            for m in range(16)
        ]
        for h in handles:
            h.wait()
        for v in range(NV):
            iq = lanes + (v * 16)
            s3 = s_v[pl.ds(v * 16, 16)]
            q2 = iq * 2
            bx = [w_v[r, pl.ds(v * 16, 16)] for r in range(4)]
            by = [w_v[4 + r, pl.ds(v * 16, 16)] for r in range(4)]
            bz = [w_v[8 + r, pl.ds(v * 16, 16)] for r in range(4)]
            d1 = {}
            d2 = {}
            for mx in range(4):
                for c in range(3):
                    u = s3 + (mx * 3 + c)
                    ge = (u >= 16).astype(jnp.int32)
                    d1[(mx, c)] = q2 + ge
                    d2[(mx, c)] = u - ge * 16
            acc = [_splat(0.0, jnp.float32) for _ in range(3)]
            for m in range(16):
                wyz = by[m & 3] * bz[m >> 2]
                for mx in range(4):
                    w = wyz * bx[mx]
                    for c in range(3):
                        gval = plsc.load_gather(
                            g_v, [_splat(m), d1[(mx, c)], d2[(mx, c)]])
                        acc[c] = acc[c] + w * gval
            for c in range(3):
                plsc.store_scatter(o_v, [iq, _splat(c)], acc[c])
        pltpu.sync_copy(o_v, out_hbm.at[pl.ds(qb, BQ)])


@jax.jit
def kernel(queries, control_points, tx, ty, tz):
    cpv = control_points.reshape(NROW, 16)
    mesh = plsc.VectorSubcoreMesh(
        core_axis_name="c", subcore_axis_name="s",
        num_cores=NCORE, num_subcores=NSUB)
    spline = pl.kernel(
        _body,
        out_type=jax.ShapeDtypeStruct((NQ, 3), jnp.float32),
        mesh=mesh,
        compiler_params=pltpu.CompilerParams(
            needs_layout_passes=False, use_tc_tiling_on_sc=False),
        scratch_types=[
            pltpu.VMEM((NKNOT,), jnp.float32),
            pltpu.VMEM((NKNOT,), jnp.float32),
            pltpu.VMEM((NKNOT,), jnp.float32),
            pltpu.VMEM((BQ, 3), jnp.float32),
            pltpu.VMEM((16, 2 * BQ), jnp.int32),
            pltpu.VMEM((16, 2 * BQ, 16), jnp.float32),
            pltpu.VMEM((12, BQ), jnp.float32),
            pltpu.VMEM((BQ,), jnp.int32),
            pltpu.VMEM((BQ, 3), jnp.float32),
            pltpu.SemaphoreType.DMA,
        ],
    )
    return spline(queries, cpv, tx, ty, tz)


# speedup vs baseline: 244.2933x; 3.3143x over previous
import functools

import jax
import jax.numpy as jnp
from jax import lax
from jax.experimental import pallas as pl
from jax.experimental.pallas import tpu as pltpu
from jax.experimental.pallas import tpu_sc as plsc

NQ = 131072
GRID = 128
NP = GRID * GRID * GRID
NSEG = GRID - 3
NKNOT = GRID + 4
NROW = NP * 3 // 16
NCORE = 2
NSUB = 16
NW = NCORE * NSUB
QPW = NQ // NW
BQ = 64
NV = BQ // 16
NSTEP = QPW // BQ
PPW = NP // NW
PK = 2048
PSTEP = PPW // PK


def _splat(v, dtype=jnp.int32):
    return jnp.full((16,), v, dtype)


def _basis(tv, x):
    xi = x * jnp.float32(NSEG)
    j0 = jnp.clip(xi.astype(jnp.int32), 0, NSEG - 1)
    ta = plsc.load_gather(tv, [j0 + 3])
    tb = plsc.load_gather(tv, [j0 + 4])
    j = j0 + (x >= tb).astype(jnp.int32) - (x < ta).astype(jnp.int32)
    j = jnp.clip(j, 0, NSEG - 1)
    t1 = plsc.load_gather(tv, [j + 1])
    t2 = plsc.load_gather(tv, [j + 2])
    t3 = plsc.load_gather(tv, [j + 3])
    t4 = plsc.load_gather(tv, [j + 4])
    t5 = plsc.load_gather(tv, [j + 5])
    t6 = plsc.load_gather(tv, [j + 6])
    l1 = x - t3
    l2 = x - t2
    l3 = x - t1
    r1 = t4 - x
    r2 = t5 - x
    r3 = t6 - x
    tmp = jnp.float32(1.0) / (r1 + l1)
    n0 = r1 * tmp
    n1 = l1 * tmp
    tmp = n0 / (r1 + l2)
    n0 = r1 * tmp
    sv = l2 * tmp
    tmp = n1 / (r2 + l1)
    n1 = sv + r2 * tmp
    n2 = l1 * tmp
    tmp = n0 / (r1 + l3)
    n0 = r1 * tmp
    sv = l3 * tmp
    tmp = n1 / (r2 + l2)
    n1 = sv + r2 * tmp
    sv = l2 * tmp
    tmp = n2 / (r3 + l1)
    n2 = sv + r3 * tmp
    n3 = l1 * tmp
    return j, n0, n1, n2, n3


def _inter_body(cpt_hbm, cpx_hbm, p_v, o_v, sem):
    cid = lax.axis_index("c")
    sid = lax.axis_index("s")
    wid = sid * NCORE + cid
    pbase = wid * PPW
    lanes3 = lax.iota(jnp.int32, 16) * 3

    @pl.loop(0, PSTEP)
    def _step(si):
        p0 = pbase + si * PK
        for c in range(3):
            pltpu.sync_copy(cpt_hbm.at[c, pl.ds(p0, PK)], p_v.at[c])
        for i in range(PK // 16):
            base = i * 48
            for c in range(3):
                v = p_v[c, pl.ds(i * 16, 16)]
                plsc.store_scatter(o_v, [lanes3 + (base + c)], v)
        pltpu.sync_copy(o_v, cpx_hbm.at[pl.ds(p0 * 3, PK * 3)])


def _body(qt_hbm, cp_hbm, tx_hbm, ty_hbm, tz_hbm, out_hbm,
          txv, tyv, tzv, q_v, idx_v, g_v, w_v, s_v, o_v, sem):
    cid = lax.axis_index("c")
    sid = lax.axis_index("s")
    wid = sid * NCORE + cid
    qbase = wid * QPW
    pltpu.sync_copy(tx_hbm, txv)
    pltpu.sync_copy(ty_hbm, tyv)
    pltpu.sync_copy(tz_hbm, tzv)
    lanes = lax.iota(jnp.int32, 16)

    @pl.loop(0, NSTEP)
    def _step(si):
        qb = qbase + si * BQ
        for c in range(3):
            pltpu.sync_copy(qt_hbm.at[c, pl.ds(qb, BQ)], q_v.at[c])
        for v in range(NV):
            iq = lanes + (v * 16)
            qx = q_v[0, pl.ds(v * 16, 16)]
            qy = q_v[1, pl.ds(v * 16, 16)]
            qz = q_v[2, pl.ds(v * 16, 16)]
            jx, bx0, bx1, bx2, bx3 = _basis(txv, qx)
            jy, by0, by1, by2, by3 = _basis(tyv, qy)
            jz, bz0, bz1, bz2, bz3 = _basis(tzv, qz)
            allb = (bx0, bx1, bx2, bx3, by0, by1, by2, by3, bz0, bz1, bz2, bz3)
            for r, n in enumerate(allb):
                w_v[r, pl.ds(v * 16, 16)] = n
            base3 = jx * 3 + jy * 384 + jz * 49152
            s_v[pl.ds(v * 16, 16)] = base3 & 15
            gbase = base3 >> 4
            q2 = iq * 2
            for m in range(16):
                my = m & 3
                mz = m >> 2
                g0 = gbase + (24 * my + 3072 * mz)
                g1 = jnp.minimum(g0 + 1, NROW - 1)
                plsc.store_scatter(idx_v, [_splat(m), q2], g0)
                plsc.store_scatter(idx_v, [_splat(m), q2 + 1], g1)
        handles = [
            pltpu.async_copy(cp_hbm.at[idx_v.at[m]], g_v.at[m], sem)
            for m in range(16)
        ]
        for h in handles:
            h.wait()
        for v in range(NV):
            iq = lanes + (v * 16)
            s3 = s_v[pl.ds(v * 16, 16)]
            q2 = iq * 2
            bx = [w_v[r, pl.ds(v * 16, 16)] for r in range(4)]
            by = [w_v[4 + r, pl.ds(v * 16, 16)] for r in range(4)]
            bz = [w_v[8 + r, pl.ds(v * 16, 16)] for r in range(4)]
            d1 = {}
            d2 = {}
            for mx in range(4):
                for c in range(3):
                    u = s3 + (mx * 3 + c)
                    ge = (u >= 16).astype(jnp.int32)
                    d1[(mx, c)] = q2 + ge
                    d2[(mx, c)] = u - ge * 16
            acc = [_splat(0.0, jnp.float32) for _ in range(3)]
            for m in range(16):
                wyz = by[m & 3] * bz[m >> 2]
                for mx in range(4):
                    w = wyz * bx[mx]
                    for c in range(3):
                        gval = plsc.load_gather(
                            g_v, [_splat(m), d1[(mx, c)], d2[(mx, c)]])
                        acc[c] = acc[c] + w * gval
            for c in range(3):
                o_v[c, pl.ds(v * 16, 16)] = acc[c]
        for c in range(3):
            pltpu.sync_copy(o_v.at[c], out_hbm.at[c, pl.ds(qb, BQ)])


@jax.jit
def kernel(queries, control_points, tx, ty, tz):
    qt = queries.T
    cpt = control_points.T
    mesh = plsc.VectorSubcoreMesh(
        core_axis_name="c", subcore_axis_name="s",
        num_cores=NCORE, num_subcores=NSUB)
    cparams = pltpu.CompilerParams(
        needs_layout_passes=False, use_tc_tiling_on_sc=False)
    interleave = pl.kernel(
        _inter_body,
        out_type=jax.ShapeDtypeStruct((NP * 3,), jnp.float32),
        mesh=mesh,
        compiler_params=cparams,
        scratch_types=[
            pltpu.VMEM((3, PK), jnp.float32),
            pltpu.VMEM((PK * 3,), jnp.float32),
            pltpu.SemaphoreType.DMA,
        ],
    )
    cpx = interleave(cpt)
    spline = pl.kernel(
        _body,
        out_type=jax.ShapeDtypeStruct((3, NQ), jnp.float32),
        mesh=mesh,
        compiler_params=cparams,
        scratch_types=[
            pltpu.VMEM((NKNOT,), jnp.float32),
            pltpu.VMEM((NKNOT,), jnp.float32),
            pltpu.VMEM((NKNOT,), jnp.float32),
            pltpu.VMEM((3, BQ), jnp.float32),
            pltpu.VMEM((16, 2 * BQ), jnp.int32),
            pltpu.VMEM((16, 2 * BQ, 16), jnp.float32),
            pltpu.VMEM((12, BQ), jnp.float32),
            pltpu.VMEM((BQ,), jnp.int32),
            pltpu.VMEM((3, BQ), jnp.float32),
            pltpu.SemaphoreType.DMA,
        ],
    )
    outt = spline(qt, cpx.reshape(NROW, 16), tx, ty, tz)
    return outt.T


# speedup vs baseline: 445.7100x; 1.8245x over previous
import functools

import jax
import jax.numpy as jnp
from jax import lax
from jax.experimental import pallas as pl
from jax.experimental.pallas import tpu as pltpu
from jax.experimental.pallas import tpu_sc as plsc

NQ = 131072
GRID = 128
NP = GRID * GRID * GRID
NSEG = GRID - 3
NKNOT = GRID + 4
NROW = NP * 3 // 16
NCORE = 2
NSUB = 16
NW = NCORE * NSUB
QPW = NQ // NW
BQ = 64
NV = BQ // 16
NSTEP = QPW // BQ
PPW = NP // NW
PK = 2048
PSTEP = PPW // PK


def _splat(v, dtype=jnp.int32):
    return jnp.full((16,), v, dtype)


def _basis(tv, x):
    xi = x * jnp.float32(NSEG)
    j0 = jnp.clip(xi.astype(jnp.int32), 0, NSEG - 1)
    ta = plsc.load_gather(tv, [j0 + 3])
    tb = plsc.load_gather(tv, [j0 + 4])
    j = j0 + (x >= tb).astype(jnp.int32) - (x < ta).astype(jnp.int32)
    j = jnp.clip(j, 0, NSEG - 1)
    t1 = plsc.load_gather(tv, [j + 1])
    t2 = plsc.load_gather(tv, [j + 2])
    t3 = plsc.load_gather(tv, [j + 3])
    t4 = plsc.load_gather(tv, [j + 4])
    t5 = plsc.load_gather(tv, [j + 5])
    t6 = plsc.load_gather(tv, [j + 6])
    l1 = x - t3
    l2 = x - t2
    l3 = x - t1
    r1 = t4 - x
    r2 = t5 - x
    r3 = t6 - x
    tmp = jnp.float32(1.0) / (r1 + l1)
    n0 = r1 * tmp
    n1 = l1 * tmp
    tmp = n0 / (r1 + l2)
    n0 = r1 * tmp
    sv = l2 * tmp
    tmp = n1 / (r2 + l1)
    n1 = sv + r2 * tmp
    n2 = l1 * tmp
    tmp = n0 / (r1 + l3)
    n0 = r1 * tmp
    sv = l3 * tmp
    tmp = n1 / (r2 + l2)
    n1 = sv + r2 * tmp
    sv = l2 * tmp
    tmp = n2 / (r3 + l1)
    n2 = sv + r3 * tmp
    n3 = l1 * tmp
    return j, n0, n1, n2, n3


def _inter_body(raw_hbm, cpx_hbm, p_v, o_v, sem):
    cid = lax.axis_index("c")
    sid = lax.axis_index("s")
    wid = sid * NCORE + cid
    pbase = wid * PPW
    lanes3 = lax.iota(jnp.int32, 16) * 3

    @pl.loop(0, PSTEP)
    def _step(si):
        p0 = pbase + si * PK
        pltpu.sync_copy(raw_hbm.at[pl.ds(p0 * 4, PK * 4)], p_v)
        for g in range(PK // 16):
            base = g * 48
            src = (g >> 3) * 512 + (g & 7) * 16
            for c in range(3):
                v = p_v[pl.ds(src + c * 128, 16)]
                plsc.store_scatter(o_v, [lanes3 + (base + c)], v)
        pltpu.sync_copy(o_v, cpx_hbm.at[pl.ds(p0 * 3, PK * 3)])


def _body(qt_hbm, cp_hbm, tx_hbm, ty_hbm, tz_hbm, out_hbm,
          txv, tyv, tzv, q_v, idx_v, g_v, w_v, s_v, o_v, sem):
    cid = lax.axis_index("c")
    sid = lax.axis_index("s")
    wid = sid * NCORE + cid
    qbase = wid * QPW
    pltpu.sync_copy(tx_hbm, txv)
    pltpu.sync_copy(ty_hbm, tyv)
    pltpu.sync_copy(tz_hbm, tzv)
    lanes = lax.iota(jnp.int32, 16)

    @pl.loop(0, NSTEP)
    def _step(si):
        qb = qbase + si * BQ
        qoff = pl.multiple_of(((qb >> 7) * 512) + (qb & 127), 64)
        for c in range(3):
            pltpu.sync_copy(qt_hbm.at[pl.ds(qoff + c * 128, BQ)], q_v.at[c])
        for v in range(NV):
            iq = lanes + (v * 16)
            qx = q_v[0, pl.ds(v * 16, 16)]
            qy = q_v[1, pl.ds(v * 16, 16)]
            qz = q_v[2, pl.ds(v * 16, 16)]
            jx, bx0, bx1, bx2, bx3 = _basis(txv, qx)
            jy, by0, by1, by2, by3 = _basis(tyv, qy)
            jz, bz0, bz1, bz2, bz3 = _basis(tzv, qz)
            allb = (bx0, bx1, bx2, bx3, by0, by1, by2, by3, bz0, bz1, bz2, bz3)
            for r, n in enumerate(allb):
                w_v[r, pl.ds(v * 16, 16)] = n
            base3 = jx * 3 + jy * 384 + jz * 49152
            s_v[pl.ds(v * 16, 16)] = base3 & 15
            gbase = base3 >> 4
            q2 = iq * 2
            for m in range(16):
                my = m & 3
                mz = m >> 2
                g0 = gbase + (24 * my + 3072 * mz)
                g1 = jnp.minimum(g0 + 1, NROW - 1)
                plsc.store_scatter(idx_v, [_splat(m), q2], g0)
                plsc.store_scatter(idx_v, [_splat(m), q2 + 1], g1)
        handles = [
            pltpu.async_copy(cp_hbm.at[idx_v.at[m]], g_v.at[m], sem)
            for m in range(16)
        ]
        for h in handles:
            h.wait()
        for v in range(NV):
            iq = lanes + (v * 16)
            s3 = s_v[pl.ds(v * 16, 16)]
            q2 = iq * 2
            bx = [w_v[r, pl.ds(v * 16, 16)] for r in range(4)]
            by = [w_v[4 + r, pl.ds(v * 16, 16)] for r in range(4)]
            bz = [w_v[8 + r, pl.ds(v * 16, 16)] for r in range(4)]
            d1 = {}
            d2 = {}
            for mx in range(4):
                for c in range(3):
                    u = s3 + (mx * 3 + c)
                    ge = (u >= 16).astype(jnp.int32)
                    d1[(mx, c)] = q2 + ge
                    d2[(mx, c)] = u - ge * 16
            acc = [_splat(0.0, jnp.float32) for _ in range(3)]
            for m in range(16):
                wyz = by[m & 3] * bz[m >> 2]
                for mx in range(4):
                    w = wyz * bx[mx]
                    for c in range(3):
                        gval = plsc.load_gather(
                            g_v, [_splat(m), d1[(mx, c)], d2[(mx, c)]])
                        acc[c] = acc[c] + w * gval
            for c in range(3):
                o_v[c, pl.ds(v * 16, 16)] = acc[c]
        for c in range(3):
            pltpu.sync_copy(o_v.at[c], out_hbm.at[pl.ds(qoff + c * 128, BQ)])


@jax.jit
def kernel(queries, control_points, tx, ty, tz):
    qt = jnp.transpose(
        jnp.pad(queries.T, ((0, 1), (0, 0))).reshape(4, NQ // 128, 128),
        (1, 0, 2)).reshape(-1)
    cpraw = jnp.transpose(
        jnp.pad(control_points.T, ((0, 1), (0, 0))).reshape(4, NP // 128, 128),
        (1, 0, 2)).reshape(-1)
    mesh = plsc.VectorSubcoreMesh(
        core_axis_name="c", subcore_axis_name="s",
        num_cores=NCORE, num_subcores=NSUB)
    cparams = pltpu.CompilerParams(
        needs_layout_passes=False, use_tc_tiling_on_sc=False)
    interleave = pl.kernel(
        _inter_body,
        out_type=jax.ShapeDtypeStruct((NP * 3,), jnp.float32),
        mesh=mesh,
        compiler_params=cparams,
        scratch_types=[
            pltpu.VMEM((PK * 4,), jnp.float32),
            pltpu.VMEM((PK * 3,), jnp.float32),
            pltpu.SemaphoreType.DMA,
        ],
    )
    cpx = interleave(cpraw)
    spline = pl.kernel(
        _body,
        out_type=jax.ShapeDtypeStruct((NQ * 4,), jnp.float32),
        mesh=mesh,
        compiler_params=cparams,
        scratch_types=[
            pltpu.VMEM((NKNOT,), jnp.float32),
            pltpu.VMEM((NKNOT,), jnp.float32),
            pltpu.VMEM((NKNOT,), jnp.float32),
            pltpu.VMEM((3, BQ), jnp.float32),
            pltpu.VMEM((16, 2 * BQ), jnp.int32),
            pltpu.VMEM((16, 2 * BQ, 16), jnp.float32),
            pltpu.VMEM((12, BQ), jnp.float32),
            pltpu.VMEM((BQ,), jnp.int32),
            pltpu.VMEM((3, BQ), jnp.float32),
            pltpu.SemaphoreType.DMA,
        ],
    )
    oraw = spline(qt, cpx.reshape(NROW, 16), tx, ty, tz)
    out = jnp.transpose(
        oraw.reshape(NQ // 128, 4, 128), (1, 0, 2)).reshape(4, NQ)[:3]
    return out.T


# speedup vs baseline: 534.6276x; 1.1995x over previous
import functools

import jax
import jax.numpy as jnp
from jax import lax
from jax.experimental import pallas as pl
from jax.experimental.pallas import tpu as pltpu
from jax.experimental.pallas import tpu_sc as plsc

NQ = 131072
GRID = 128
NP = GRID * GRID * GRID
NSEG = GRID - 3
NKNOT = GRID + 4
NROW = NP * 3 // 16
NCORE = 2
NSUB = 16
NW = NCORE * NSUB
QPW = NQ // NW
BQ = 64
NV = BQ // 16
NSTEP = QPW // BQ
PPW = NP // NW
PK = 2048
PSTEP = PPW // PK


def _splat(v, dtype=jnp.int32):
    return jnp.full((16,), v, dtype)


def _basis(tv, x):
    xi = x * jnp.float32(NSEG)
    j0 = jnp.clip(xi.astype(jnp.int32), 0, NSEG - 1)
    ta = plsc.load_gather(tv, [j0 + 3])
    tb = plsc.load_gather(tv, [j0 + 4])
    j = j0 + (x >= tb).astype(jnp.int32) - (x < ta).astype(jnp.int32)
    j = jnp.clip(j, 0, NSEG - 1)
    t1 = plsc.load_gather(tv, [j + 1])
    t2 = plsc.load_gather(tv, [j + 2])
    t3 = plsc.load_gather(tv, [j + 3])
    t4 = plsc.load_gather(tv, [j + 4])
    t5 = plsc.load_gather(tv, [j + 5])
    t6 = plsc.load_gather(tv, [j + 6])
    l1 = x - t3
    l2 = x - t2
    l3 = x - t1
    r1 = t4 - x
    r2 = t5 - x
    r3 = t6 - x
    tmp = jnp.float32(1.0) / (r1 + l1)
    n0 = r1 * tmp
    n1 = l1 * tmp
    tmp = n0 / (r1 + l2)
    n0 = r1 * tmp
    sv = l2 * tmp
    tmp = n1 / (r2 + l1)
    n1 = sv + r2 * tmp
    n2 = l1 * tmp
    tmp = n0 / (r1 + l3)
    n0 = r1 * tmp
    sv = l3 * tmp
    tmp = n1 / (r2 + l2)
    n1 = sv + r2 * tmp
    sv = l2 * tmp
    tmp = n2 / (r3 + l1)
    n2 = sv + r3 * tmp
    n3 = l1 * tmp
    return j, n0, n1, n2, n3


def _inter_body(raw_hbm, cpx_hbm, p_v, o_v, sem):
    cid = lax.axis_index("c")
    sid = lax.axis_index("s")
    wid = sid * NCORE + cid
    pbase = wid * PPW
    lanes3 = lax.iota(jnp.int32, 16) * 3

    @pl.loop(0, PSTEP)
    def _step(si):
        p0 = pbase + si * PK
        pltpu.sync_copy(raw_hbm.at[pl.ds(p0 * 4, PK * 4)], p_v)
        for g in range(PK // 16):
            base = g * 48
            src = (g >> 3) * 512 + (g & 7) * 16
            for c in range(3):
                v = p_v[pl.ds(src + c * 128, 16)]
                plsc.store_scatter(o_v, [lanes3 + (base + c)], v)
        pltpu.sync_copy(o_v, cpx_hbm.at[pl.ds(p0 * 3, PK * 3)])


def _body(qt_hbm, cp_hbm, tx_hbm, ty_hbm, tz_hbm, out_hbm,
          txv, tyv, tzv, q_v, idx_a, idx_b, g_a, g_b, w_a, w_b,
          s_a, s_b, o_v, sem_a, sem_b):
    cid = lax.axis_index("c")
    sid = lax.axis_index("s")
    wid = sid * NCORE + cid
    qbase = wid * QPW
    pltpu.sync_copy(tx_hbm, txv)
    pltpu.sync_copy(ty_hbm, tyv)
    pltpu.sync_copy(tz_hbm, tzv)
    lanes = lax.iota(jnp.int32, 16)

    def phase1(si, idx_v, w_v, s_v):
        qb = qbase + si * BQ
        qoff = pl.multiple_of(((qb >> 7) * 512) + (qb & 127), 64)
        for c in range(3):
            pltpu.sync_copy(qt_hbm.at[pl.ds(qoff + c * 128, BQ)], q_v.at[c])
        for v in range(NV):
            iq = lanes + (v * 16)
            qx = q_v[0, pl.ds(v * 16, 16)]
            qy = q_v[1, pl.ds(v * 16, 16)]
            qz = q_v[2, pl.ds(v * 16, 16)]
            jx, bx0, bx1, bx2, bx3 = _basis(txv, qx)
            jy, by0, by1, by2, by3 = _basis(tyv, qy)
            jz, bz0, bz1, bz2, bz3 = _basis(tzv, qz)
            allb = (bx0, bx1, bx2, bx3, by0, by1, by2, by3, bz0, bz1, bz2, bz3)
            for r, n in enumerate(allb):
                w_v[r, pl.ds(v * 16, 16)] = n
            base3 = jx * 3 + jy * 384 + jz * 49152
            s_v[pl.ds(v * 16, 16)] = base3 & 15
            gbase = base3 >> 4
            q2 = iq * 2
            for m in range(16):
                my = m & 3
                mz = m >> 2
                g0 = gbase + (24 * my + 3072 * mz)
                g1 = jnp.minimum(g0 + 1, NROW - 1)
                plsc.store_scatter(idx_v, [_splat(m), q2], g0)
                plsc.store_scatter(idx_v, [_splat(m), q2 + 1], g1)

    def fire(idx_v, g_v, sem):
        for m in range(16):
            pltpu.async_copy(cp_hbm.at[idx_v.at[m]], g_v.at[m], sem)

    def drain(idx_v, g_v, sem):
        for m in range(16):
            pltpu.make_async_copy(
                cp_hbm.at[idx_v.at[m]], g_v.at[m], sem).wait()

    def reduce(si, g_v, w_v, s_v):
        qb = qbase + si * BQ
        qoff = pl.multiple_of(((qb >> 7) * 512) + (qb & 127), 64)
        for v in range(NV):
            iq = lanes + (v * 16)
            s3 = s_v[pl.ds(v * 16, 16)]
            q2 = iq * 2
            bx = [w_v[r, pl.ds(v * 16, 16)] for r in range(4)]
            by = [w_v[4 + r, pl.ds(v * 16, 16)] for r in range(4)]
            bz = [w_v[8 + r, pl.ds(v * 16, 16)] for r in range(4)]
            d1 = {}
            d2 = {}
            for mx in range(4):
                for c in range(3):
                    u = s3 + (mx * 3 + c)
                    ge = (u >= 16).astype(jnp.int32)
                    d1[(mx, c)] = q2 + ge
                    d2[(mx, c)] = u - ge * 16
            acc = [_splat(0.0, jnp.float32) for _ in range(3)]
            for m in range(16):
                wyz = by[m & 3] * bz[m >> 2]
                for mx in range(4):
                    w = wyz * bx[mx]
                    for c in range(3):
                        gval = plsc.load_gather(
                            g_v, [_splat(m), d1[(mx, c)], d2[(mx, c)]])
                        acc[c] = acc[c] + w * gval
            for c in range(3):
                o_v[c, pl.ds(v * 16, 16)] = acc[c]
        for c in range(3):
            pltpu.sync_copy(o_v.at[c], out_hbm.at[pl.ds(qoff + c * 128, BQ)])

    phase1(0, idx_a, w_a, s_a)
    fire(idx_a, g_a, sem_a)

    @pl.loop(0, NSTEP // 2)
    def _steps(i):
        s0 = i * 2
        phase1(s0 + 1, idx_b, w_b, s_b)
        fire(idx_b, g_b, sem_b)
        drain(idx_a, g_a, sem_a)
        reduce(s0, g_a, w_a, s_a)

        @pl.when(i < NSTEP // 2 - 1)
        def _():
            phase1(s0 + 2, idx_a, w_a, s_a)
            fire(idx_a, g_a, sem_a)

        drain(idx_b, g_b, sem_b)
        reduce(s0 + 1, g_b, w_b, s_b)


@jax.jit
def kernel(queries, control_points, tx, ty, tz):
    qt = jnp.transpose(
        jnp.pad(queries.T, ((0, 1), (0, 0))).reshape(4, NQ // 128, 128),
        (1, 0, 2)).reshape(-1)
    cpraw = jnp.transpose(
        jnp.pad(control_points.T, ((0, 1), (0, 0))).reshape(4, NP // 128, 128),
        (1, 0, 2)).reshape(-1)
    mesh = plsc.VectorSubcoreMesh(
        core_axis_name="c", subcore_axis_name="s",
        num_cores=NCORE, num_subcores=NSUB)
    cparams = pltpu.CompilerParams(
        needs_layout_passes=False, use_tc_tiling_on_sc=False)
    interleave = pl.kernel(
        _inter_body,
        out_type=jax.ShapeDtypeStruct((NP * 3,), jnp.float32),
        mesh=mesh,
        compiler_params=cparams,
        scratch_types=[
            pltpu.VMEM((PK * 4,), jnp.float32),
            pltpu.VMEM((PK * 3,), jnp.float32),
            pltpu.SemaphoreType.DMA,
        ],
    )
    cpx = interleave(cpraw)
    spline = pl.kernel(
        _body,
        out_type=jax.ShapeDtypeStruct((NQ * 4,), jnp.float32),
        mesh=mesh,
        compiler_params=cparams,
        scratch_types=[
            pltpu.VMEM((NKNOT,), jnp.float32),
            pltpu.VMEM((NKNOT,), jnp.float32),
            pltpu.VMEM((NKNOT,), jnp.float32),
            pltpu.VMEM((3, BQ), jnp.float32),
            pltpu.VMEM((16, 2 * BQ), jnp.int32),
            pltpu.VMEM((16, 2 * BQ), jnp.int32),
            pltpu.VMEM((16, 2 * BQ, 16), jnp.float32),
            pltpu.VMEM((16, 2 * BQ, 16), jnp.float32),
            pltpu.VMEM((12, BQ), jnp.float32),
            pltpu.VMEM((12, BQ), jnp.float32),
            pltpu.VMEM((BQ,), jnp.int32),
            pltpu.VMEM((BQ,), jnp.int32),
            pltpu.VMEM((3, BQ), jnp.float32),
            pltpu.SemaphoreType.DMA,
            pltpu.SemaphoreType.DMA,
        ],
    )
    oraw = spline(qt, cpx.reshape(NROW, 16), tx, ty, tz)
    out = jnp.transpose(
        oraw.reshape(NQ // 128, 4, 128), (1, 0, 2)).reshape(4, NQ)[:3]
    return out.T


# speedup vs baseline: 550.6312x; 1.0299x over previous
import functools

import jax
import jax.numpy as jnp
from jax import lax
from jax.experimental import pallas as pl
from jax.experimental.pallas import tpu as pltpu
from jax.experimental.pallas import tpu_sc as plsc

NQ = 131072
GRID = 128
NP = GRID * GRID * GRID
NSEG = GRID - 3
NKNOT = GRID + 4
NROW = NP * 3 // 16
NCORE = 2
NSUB = 16
NW = NCORE * NSUB
QPW = NQ // NW
BQ = 64
NV = BQ // 16
NSTEP = QPW // BQ
PPW = NP // NW
PK = 8192
PSTEP = PPW // PK


def _splat(v, dtype=jnp.int32):
    return jnp.full((16,), v, dtype)


def _basis(tv, x):
    xi = x * jnp.float32(NSEG)
    j0 = jnp.clip(xi.astype(jnp.int32), 0, NSEG - 1)
    ta = plsc.load_gather(tv, [j0 + 3])
    tb = plsc.load_gather(tv, [j0 + 4])
    j = j0 + (x >= tb).astype(jnp.int32) - (x < ta).astype(jnp.int32)
    j = jnp.clip(j, 0, NSEG - 1)
    t1 = plsc.load_gather(tv, [j + 1])
    t2 = plsc.load_gather(tv, [j + 2])
    t3 = plsc.load_gather(tv, [j + 3])
    t4 = plsc.load_gather(tv, [j + 4])
    t5 = plsc.load_gather(tv, [j + 5])
    t6 = plsc.load_gather(tv, [j + 6])
    l1 = x - t3
    l2 = x - t2
    l3 = x - t1
    r1 = t4 - x
    r2 = t5 - x
    r3 = t6 - x
    tmp = jnp.float32(1.0) / (r1 + l1)
    n0 = r1 * tmp
    n1 = l1 * tmp
    tmp = n0 / (r1 + l2)
    n0 = r1 * tmp
    sv = l2 * tmp
    tmp = n1 / (r2 + l1)
    n1 = sv + r2 * tmp
    n2 = l1 * tmp
    tmp = n0 / (r1 + l3)
    n0 = r1 * tmp
    sv = l3 * tmp
    tmp = n1 / (r2 + l2)
    n1 = sv + r2 * tmp
    sv = l2 * tmp
    tmp = n2 / (r3 + l1)
    n2 = sv + r3 * tmp
    n3 = l1 * tmp
    return j, n0, n1, n2, n3


def _inter_body(raw_hbm, cpx_hbm, p_v, o_v, sem):
    cid = lax.axis_index("c")
    sid = lax.axis_index("s")
    wid = sid * NCORE + cid
    pbase = wid * PPW
    lanes3 = lax.iota(jnp.int32, 16) * 3

    @pl.loop(0, PSTEP)
    def _step(si):
        p0 = pbase + si * PK
        pltpu.sync_copy(raw_hbm.at[pl.ds(p0 * 4, PK * 4)], p_v)
        for g in range(PK // 16):
            base = g * 48
            src = (g >> 3) * 512 + (g & 7) * 16
            for c in range(3):
                v = p_v[pl.ds(src + c * 128, 16)]
                plsc.store_scatter(o_v, [lanes3 + (base + c)], v)
        pltpu.sync_copy(o_v, cpx_hbm.at[pl.ds(p0 * 3, PK * 3)])


def _body(qt_hbm, cp_hbm, tx_hbm, ty_hbm, tz_hbm, out_hbm,
          txv, tyv, tzv, q_v, idx_a, idx_b, g_a, g_b, w_a, w_b,
          s_a, s_b, o_v, sem_a, sem_b):
    cid = lax.axis_index("c")
    sid = lax.axis_index("s")
    wid = sid * NCORE + cid
    qbase = wid * QPW
    pltpu.sync_copy(tx_hbm, txv)
    pltpu.sync_copy(ty_hbm, tyv)
    pltpu.sync_copy(tz_hbm, tzv)
    lanes = lax.iota(jnp.int32, 16)

    def phase1(si, idx_v, w_v, s_v):
        qb = qbase + si * BQ
        qoff = pl.multiple_of(((qb >> 7) * 512) + (qb & 127), 64)
        for c in range(3):
            pltpu.sync_copy(qt_hbm.at[pl.ds(qoff + c * 128, BQ)], q_v.at[c])
        for v in range(NV):
            iq = lanes + (v * 16)
            qx = q_v[0, pl.ds(v * 16, 16)]
            qy = q_v[1, pl.ds(v * 16, 16)]
            qz = q_v[2, pl.ds(v * 16, 16)]
            jx, bx0, bx1, bx2, bx3 = _basis(txv, qx)
            jy, by0, by1, by2, by3 = _basis(tyv, qy)
            jz, bz0, bz1, bz2, bz3 = _basis(tzv, qz)
            allb = (bx0, bx1, bx2, bx3, by0, by1, by2, by3, bz0, bz1, bz2, bz3)
            for r, n in enumerate(allb):
                w_v[r, pl.ds(v * 16, 16)] = n
            base3 = jx * 3 + jy * 384 + jz * 49152
            s_v[pl.ds(v * 16, 16)] = base3 & 15
            gbase = base3 >> 4
            q2 = iq * 2
            for m in range(16):
                my = m & 3
                mz = m >> 2
                g0 = gbase + (24 * my + 3072 * mz)
                g1 = jnp.minimum(g0 + 1, NROW - 1)
                plsc.store_scatter(idx_v, [_splat(m), q2], g0)
                plsc.store_scatter(idx_v, [_splat(m), q2 + 1], g1)

    def fire(idx_v, g_v, sem):
        for m in range(16):
            pltpu.async_copy(cp_hbm.at[idx_v.at[m]], g_v.at[m], sem)

    def drain(idx_v, g_v, sem):
        for m in range(16):
            pltpu.make_async_copy(
                cp_hbm.at[idx_v.at[m]], g_v.at[m], sem).wait()

    def reduce(si, g_v, w_v, s_v):
        qb = qbase + si * BQ
        qoff = pl.multiple_of(((qb >> 7) * 512) + (qb & 127), 64)
        for v in range(NV):
            iq = lanes + (v * 16)
            s3 = s_v[pl.ds(v * 16, 16)]
            q2 = iq * 2
            bx = [w_v[r, pl.ds(v * 16, 16)] for r in range(4)]
            by = [w_v[4 + r, pl.ds(v * 16, 16)] for r in range(4)]
            bz = [w_v[8 + r, pl.ds(v * 16, 16)] for r in range(4)]
            d1 = {}
            d2 = {}
            for mx in range(4):
                for c in range(3):
                    u = s3 + (mx * 3 + c)
                    ge = (u >= 16).astype(jnp.int32)
                    d1[(mx, c)] = q2 + ge
                    d2[(mx, c)] = u - ge * 16
            acc = [_splat(0.0, jnp.float32) for _ in range(3)]
            for m in range(16):
                wyz = by[m & 3] * bz[m >> 2]
                for mx in range(4):
                    w = wyz * bx[mx]
                    for c in range(3):
                        gval = plsc.load_gather(
                            g_v, [_splat(m), d1[(mx, c)], d2[(mx, c)]])
                        acc[c] = acc[c] + w * gval
            for c in range(3):
                o_v[c, pl.ds(v * 16, 16)] = acc[c]
        for c in range(3):
            pltpu.sync_copy(o_v.at[c], out_hbm.at[pl.ds(qoff + c * 128, BQ)])

    phase1(0, idx_a, w_a, s_a)
    fire(idx_a, g_a, sem_a)

    @pl.loop(0, NSTEP // 2)
    def _steps(i):
        s0 = i * 2
        phase1(s0 + 1, idx_b, w_b, s_b)
        fire(idx_b, g_b, sem_b)
        drain(idx_a, g_a, sem_a)
        reduce(s0, g_a, w_a, s_a)

        @pl.when(i < NSTEP // 2 - 1)
        def _():
            phase1(s0 + 2, idx_a, w_a, s_a)
            fire(idx_a, g_a, sem_a)

        drain(idx_b, g_b, sem_b)
        reduce(s0 + 1, g_b, w_b, s_b)


@jax.jit
def kernel(queries, control_points, tx, ty, tz):
    qt = jnp.transpose(
        jnp.pad(queries.T, ((0, 1), (0, 0))).reshape(4, NQ // 128, 128),
        (1, 0, 2)).reshape(-1)
    cpraw = jnp.transpose(
        jnp.pad(control_points.T, ((0, 1), (0, 0))).reshape(4, NP // 128, 128),
        (1, 0, 2)).reshape(-1)
    mesh = plsc.VectorSubcoreMesh(
        core_axis_name="c", subcore_axis_name="s",
        num_cores=NCORE, num_subcores=NSUB)
    cparams = pltpu.CompilerParams(
        needs_layout_passes=False, use_tc_tiling_on_sc=False)
    interleave = pl.kernel(
        _inter_body,
        out_type=jax.ShapeDtypeStruct((NP * 3,), jnp.float32),
        mesh=mesh,
        compiler_params=cparams,
        scratch_types=[
            pltpu.VMEM((PK * 4,), jnp.float32),
            pltpu.VMEM((PK * 3,), jnp.float32),
            pltpu.SemaphoreType.DMA,
        ],
    )
    cpx = interleave(cpraw)
    spline = pl.kernel(
        _body,
        out_type=jax.ShapeDtypeStruct((NQ * 4,), jnp.float32),
        mesh=mesh,
        compiler_params=cparams,
        scratch_types=[
            pltpu.VMEM((NKNOT,), jnp.float32),
            pltpu.VMEM((NKNOT,), jnp.float32),
            pltpu.VMEM((NKNOT,), jnp.float32),
            pltpu.VMEM((3, BQ), jnp.float32),
            pltpu.VMEM((16, 2 * BQ), jnp.int32),
            pltpu.VMEM((16, 2 * BQ), jnp.int32),
            pltpu.VMEM((16, 2 * BQ, 16), jnp.float32),
            pltpu.VMEM((16, 2 * BQ, 16), jnp.float32),
            pltpu.VMEM((12, BQ), jnp.float32),
            pltpu.VMEM((12, BQ), jnp.float32),
            pltpu.VMEM((BQ,), jnp.int32),
            pltpu.VMEM((BQ,), jnp.int32),
            pltpu.VMEM((3, BQ), jnp.float32),
            pltpu.SemaphoreType.DMA,
            pltpu.SemaphoreType.DMA,
        ],
    )
    oraw = spline(qt, cpx.reshape(NROW, 16), tx, ty, tz)
    out = jnp.transpose(
        oraw.reshape(NQ // 128, 4, 128), (1, 0, 2)).reshape(4, NQ)[:3]
    return out.T


# speedup vs baseline: 552.2668x; 1.0030x over previous
import jax
import jax.numpy as jnp
from jax import lax
from jax.experimental import pallas as pl
from jax.experimental.pallas import tpu as pltpu
from jax.experimental.pallas import tpu_sc as plsc

NQ = 131072
GRID = 128
NP = GRID * GRID * GRID
NSEG = GRID - 3
NKNOT = GRID + 4
NROW = NP * 3 // 16
NCORE = 2
NSUB = 16
NW = NCORE * NSUB
QPW = NQ // NW
BQ = 64
NV = BQ // 16
NSTEP = QPW // BQ
PPW = NP // NW
PK = 8192
PSTEP = PPW // PK


def _splat(v, dtype=jnp.int32):
    return jnp.full((16,), v, dtype)


def _basis(tv, x):
    xi = x * jnp.float32(NSEG)
    j0 = jnp.clip(xi.astype(jnp.int32), 0, NSEG - 1)
    ta = plsc.load_gather(tv, [j0 + 3])
    tb = plsc.load_gather(tv, [j0 + 4])
    j = j0 + (x >= tb).astype(jnp.int32) - (x < ta).astype(jnp.int32)
    j = jnp.clip(j, 0, NSEG - 1)
    t1 = plsc.load_gather(tv, [j + 1])
    t2 = plsc.load_gather(tv, [j + 2])
    t3 = plsc.load_gather(tv, [j + 3])
    t4 = plsc.load_gather(tv, [j + 4])
    t5 = plsc.load_gather(tv, [j + 5])
    t6 = plsc.load_gather(tv, [j + 6])
    l1 = x - t3
    l2 = x - t2
    l3 = x - t1
    r1 = t4 - x
    r2 = t5 - x
    r3 = t6 - x
    tmp = jnp.float32(1.0) / (r1 + l1)
    n0 = r1 * tmp
    n1 = l1 * tmp
    tmp = n0 / (r1 + l2)
    n0 = r1 * tmp
    sv = l2 * tmp
    tmp = n1 / (r2 + l1)
    n1 = sv + r2 * tmp
    n2 = l1 * tmp
    tmp = n0 / (r1 + l3)
    n0 = r1 * tmp
    sv = l3 * tmp
    tmp = n1 / (r2 + l2)
    n1 = sv + r2 * tmp
    sv = l2 * tmp
    tmp = n2 / (r3 + l1)
    n2 = sv + r3 * tmp
    n3 = l1 * tmp
    return j, n0, n1, n2, n3


def _inter_body(raw_hbm, cpx_hbm, p_v, o_v, sem):
    cid = lax.axis_index("c")
    sid = lax.axis_index("s")
    wid = sid * NCORE + cid
    pbase = wid * PPW
    lanes3 = lax.iota(jnp.int32, 16) * 3

    @pl.loop(0, PSTEP)
    def _step(si):
        p0 = pbase + si * PK
        pltpu.sync_copy(raw_hbm.at[pl.ds(p0 * 4, PK * 4)], p_v)
        for g in range(PK // 16):
            base = g * 48
            src = (g >> 3) * 512 + (g & 7) * 16
            for c in range(3):
                v = p_v[pl.ds(src + c * 128, 16)]
                plsc.store_scatter(o_v, [lanes3 + (base + c)], v)
        pltpu.sync_copy(o_v, cpx_hbm.at[pl.ds(p0 * 3, PK * 3)])


def _body(qt_hbm, cp_hbm, tx_hbm, ty_hbm, tz_hbm, out_hbm,
          txv, tyv, tzv, q_v, idx_a, idx_b, g_a, g_b, w_a, w_b,
          s_a, s_b, o_v, sem_a, sem_b):
    cid = lax.axis_index("c")
    sid = lax.axis_index("s")
    wid = sid * NCORE + cid
    qbase = wid * QPW
    pltpu.sync_copy(tx_hbm, txv)
    pltpu.sync_copy(ty_hbm, tyv)
    pltpu.sync_copy(tz_hbm, tzv)
    lanes = lax.iota(jnp.int32, 16)

    def phase1(si, idx_v, w_v, s_v):
        qb = qbase + si * BQ
        qoff = pl.multiple_of(((qb >> 7) * 512) + (qb & 127), 64)
        for c in range(3):
            pltpu.sync_copy(qt_hbm.at[pl.ds(qoff + c * 128, BQ)], q_v.at[c])
        for v in range(NV):
            iq = lanes + (v * 16)
            qx = q_v[0, pl.ds(v * 16, 16)]
            qy = q_v[1, pl.ds(v * 16, 16)]
            qz = q_v[2, pl.ds(v * 16, 16)]
            jx, bx0, bx1, bx2, bx3 = _basis(txv, qx)
            jy, by0, by1, by2, by3 = _basis(tyv, qy)
            jz, bz0, bz1, bz2, bz3 = _basis(tzv, qz)
            allb = (bx0, bx1, bx2, bx3, by0, by1, by2, by3, bz0, bz1, bz2, bz3)
            for r, n in enumerate(allb):
                w_v[r, pl.ds(v * 16, 16)] = n
            base3 = jx * 3 + jy * 384 + jz * 49152
            s_v[pl.ds(v * 16, 16)] = base3 & 15
            gbase = base3 >> 4
            q2 = iq * 2
            for m in range(16):
                my = m & 3
                mz = m >> 2
                g0 = gbase + (24 * my + 3072 * mz)
                g1 = jnp.minimum(g0 + 1, NROW - 1)
                plsc.store_scatter(idx_v, [_splat(m), q2], g0)
                plsc.store_scatter(idx_v, [_splat(m), q2 + 1], g1)

    def fire(idx_v, g_v, sem):
        for m in range(16):
            pltpu.async_copy(cp_hbm.at[idx_v.at[m]], g_v.at[m], sem)

    def drain(idx_v, g_v, sem):
        for m in range(16):
            pltpu.make_async_copy(
                cp_hbm.at[idx_v.at[m]], g_v.at[m], sem).wait()

    def reduce(si, g_v, w_v, s_v):
        qb = qbase + si * BQ
        qoff = pl.multiple_of(((qb >> 7) * 512) + (qb & 127), 64)
        for v in range(NV):
            iq = lanes + (v * 16)
            s3 = s_v[pl.ds(v * 16, 16)]
            q2 = iq * 2
            bx = [w_v[r, pl.ds(v * 16, 16)] for r in range(4)]
            by = [w_v[4 + r, pl.ds(v * 16, 16)] for r in range(4)]
            bz = [w_v[8 + r, pl.ds(v * 16, 16)] for r in range(4)]
            d1 = {}
            d2 = {}
            for mx in range(4):
                for c in range(3):
                    u = s3 + (mx * 3 + c)
                    ge = (u >= 16).astype(jnp.int32)
                    d1[(mx, c)] = q2 + ge
                    d2[(mx, c)] = u - ge * 16
            acc = [_splat(0.0, jnp.float32) for _ in range(3)]
            for m in range(16):
                wyz = by[m & 3] * bz[m >> 2]
                for mx in range(4):
                    w = wyz * bx[mx]
                    for c in range(3):
                        gval = plsc.load_gather(
                            g_v, [_splat(m), d1[(mx, c)], d2[(mx, c)]])
                        acc[c] = acc[c] + w * gval
            for c in range(3):
                o_v[c, pl.ds(v * 16, 16)] = acc[c]
        for c in range(3):
            pltpu.sync_copy(o_v.at[c], out_hbm.at[pl.ds(qoff + c * 128, BQ)])

    phase1(0, idx_a, w_a, s_a)
    fire(idx_a, g_a, sem_a)

    @pl.loop(0, NSTEP // 2)
    def _steps(i):
        s0 = i * 2
        phase1(s0 + 1, idx_b, w_b, s_b)
        fire(idx_b, g_b, sem_b)
        drain(idx_a, g_a, sem_a)
        reduce(s0, g_a, w_a, s_a)

        @pl.when(i < NSTEP // 2 - 1)
        def _():
            phase1(s0 + 2, idx_a, w_a, s_a)
            fire(idx_a, g_a, sem_a)

        drain(idx_b, g_b, sem_b)
        reduce(s0 + 1, g_b, w_b, s_b)


@jax.jit
def kernel(queries, control_points, tx, ty, tz):
    qt = jnp.transpose(
        jnp.pad(queries.T, ((0, 1), (0, 0))).reshape(4, NQ // 128, 128),
        (1, 0, 2)).reshape(-1)
    cpraw = jnp.transpose(
        jnp.pad(control_points.T, ((0, 1), (0, 0))).reshape(4, NP // 128, 128),
        (1, 0, 2)).reshape(-1)
    mesh = plsc.VectorSubcoreMesh(
        core_axis_name="c", subcore_axis_name="s",
        num_cores=NCORE, num_subcores=NSUB)
    cparams = pltpu.CompilerParams(
        needs_layout_passes=False, use_tc_tiling_on_sc=False)
    interleave = pl.kernel(
        _inter_body,
        out_type=jax.ShapeDtypeStruct((NP * 3,), jnp.float32),
        mesh=mesh,
        compiler_params=cparams,
        scratch_types=[
            pltpu.VMEM((PK * 4,), jnp.float32),
            pltpu.VMEM((PK * 3,), jnp.float32),
            pltpu.SemaphoreType.DMA,
        ],
    )
    cpx = interleave(cpraw)
    spline = pl.kernel(
        _body,
        out_type=jax.ShapeDtypeStruct((NQ * 4,), jnp.float32),
        mesh=mesh,
        compiler_params=cparams,
        scratch_types=[
            pltpu.VMEM((NKNOT,), jnp.float32),
            pltpu.VMEM((NKNOT,), jnp.float32),
            pltpu.VMEM((NKNOT,), jnp.float32),
            pltpu.VMEM((3, BQ), jnp.float32),
            pltpu.VMEM((16, 2 * BQ), jnp.int32),
            pltpu.VMEM((16, 2 * BQ), jnp.int32),
            pltpu.VMEM((16, 2 * BQ, 16), jnp.float32),
            pltpu.VMEM((16, 2 * BQ, 16), jnp.float32),
            pltpu.VMEM((12, BQ), jnp.float32),
            pltpu.VMEM((12, BQ), jnp.float32),
            pltpu.VMEM((BQ,), jnp.int32),
            pltpu.VMEM((BQ,), jnp.int32),
            pltpu.VMEM((3, BQ), jnp.float32),
            pltpu.SemaphoreType.DMA,
            pltpu.SemaphoreType.DMA,
        ],
    )
    oraw = spline(qt, cpx.reshape(NROW, 16), tx, ty, tz)
    out = jnp.transpose(
        oraw.reshape(NQ // 128, 4, 128), (1, 0, 2)).reshape(4, NQ)[:3]
    return out.T
